# baseline (device time: 9655 ns/iter reference)
import jax
import jax.numpy as jnp
from jax import lax
from jax.experimental import pallas as pl
from jax.experimental.pallas import tpu as pltpu

BM = 256


def kernel(x, dy, gamma):
    m, d = x.shape
    nc = m // BM
    del gamma

    def body(x_hbm, dy_hbm, out_hbm, xbuf, dybuf, acc_ref,
             recv_ref, copy_sems, out_sem, send_sem, recv_sem):
        my_x = lax.axis_index("x")
        my_y = lax.axis_index("y")
        my_z = lax.axis_index("z")
        peer = (1 - my_x, my_y, my_z)

        barrier_sem = pltpu.get_barrier_semaphore()
        pl.semaphore_signal(
            barrier_sem, inc=1,
            device_id=peer, device_id_type=pl.DeviceIdType.MESH,
        )

        def chunk_copies(i, slot):
            cx = pltpu.make_async_copy(
                x_hbm.at[pl.ds(i * BM, BM), :], xbuf.at[slot],
                copy_sems.at[slot, 0])
            cy = pltpu.make_async_copy(
                dy_hbm.at[pl.ds(i * BM, BM), :], dybuf.at[slot],
                copy_sems.at[slot, 1])
            return cx, cy

        cx, cy = chunk_copies(0, 0)
        cx.start()
        cy.start()
        for i in range(nc):
            slot = i % 2
            if i + 1 < nc:
                nx, ny = chunk_copies(i + 1, (i + 1) % 2)
                nx.start()
                ny.start()
            wx, wy = chunk_copies(i, slot)
            wx.wait()
            wy.wait()

            xv = xbuf[slot]
            dyv = dybuf[slot]
            mu = jnp.mean(xv, axis=1, keepdims=True)
            xc = xv - mu
            var = jnp.mean(xc * xc, axis=1, keepdims=True)
            rstd = lax.rsqrt(var + 1e-5)
            q = dyv * xc * rstd

            ones_row = jnp.ones((1, BM), jnp.float32)
            dg = jax.lax.dot_general(
                ones_row, q, (((1,), (0,)), ((), ())),
                preferred_element_type=jnp.float32)
            db = jax.lax.dot_general(
                ones_row, dyv, (((1,), (0,)), ((), ())),
                preferred_element_type=jnp.float32)
            if i == 0:
                acc_ref[0:1, :] = dg
                acc_ref[1:2, :] = db
            else:
                acc_ref[0:1, :] = acc_ref[0:1, :] + dg
                acc_ref[1:2, :] = acc_ref[1:2, :] + db

        pl.semaphore_wait(barrier_sem, 1)
        rdma = pltpu.make_async_remote_copy(
            src_ref=acc_ref,
            dst_ref=recv_ref,
            send_sem=send_sem,
            recv_sem=recv_sem,
            device_id=peer,
            device_id_type=pl.DeviceIdType.MESH,
        )
        rdma.start()
        rdma.wait()

        acc_ref[:, :] = acc_ref[:, :] + recv_ref[:, :]
        out_copy = pltpu.make_async_copy(acc_ref, out_hbm, out_sem)
        out_copy.start()
        out_copy.wait()

    x = pltpu.with_memory_space_constraint(x, pltpu.MemorySpace.HBM)
    dy = pltpu.with_memory_space_constraint(dy, pltpu.MemorySpace.HBM)

    return pl.pallas_call(
        body,
        out_shape=jax.ShapeDtypeStruct((2, d), jnp.float32),
        in_specs=[
            pl.BlockSpec(memory_space=pltpu.MemorySpace.HBM),
            pl.BlockSpec(memory_space=pltpu.MemorySpace.HBM),
        ],
        out_specs=pl.BlockSpec(memory_space=pltpu.MemorySpace.HBM),
        scratch_shapes=[
            pltpu.VMEM((2, BM, d), jnp.float32),
            pltpu.VMEM((2, BM, d), jnp.float32),
            pltpu.VMEM((2, d), jnp.float32),
            pltpu.VMEM((2, d), jnp.float32),
            pltpu.SemaphoreType.DMA((2, 2)),
            pltpu.SemaphoreType.DMA,
            pltpu.SemaphoreType.DMA,
            pltpu.SemaphoreType.DMA,
        ],
        compiler_params=pltpu.CompilerParams(collective_id=0),
    )(x, dy)


# device time: 9115 ns/iter; 1.0592x vs baseline; 1.0592x over previous
import jax
import jax.numpy as jnp
from jax import lax
from jax.experimental import pallas as pl
from jax.experimental.pallas import tpu as pltpu

_OFFSETS = [(dx, dy_, dz) for dx in (0, 1) for dy_ in (0, 1) for dz in (0, 1)
            if (dx, dy_, dz) != (0, 0, 0)]


def kernel(x, dy, gamma):
    m, d = x.shape
    del gamma
    qm = m // 4

    def body(x_hbm, dy_hbm, out_hbm, xbuf, dybuf, acc_ref, recv_ref,
             total_ref, copy_sems, out_sem, send_sems, recv_sems):
        my_x = lax.axis_index("x")
        my_y = lax.axis_index("y")
        my_z = lax.axis_index("z")

        def peer_of(o):
            dx, dy_, dz = o
            return (1 - my_x if dx else my_x,
                    1 - my_y if dy_ else my_y,
                    1 - my_z if dz else my_z)

        barrier_sem = pltpu.get_barrier_semaphore()
        for o in _OFFSETS:
            pl.semaphore_signal(
                barrier_sem, inc=1,
                device_id=peer_of(o), device_id_type=pl.DeviceIdType.MESH,
            )

        row0 = (my_y * 2 + my_z) * qm
        cx = pltpu.make_async_copy(
            x_hbm.at[pl.ds(row0, qm), :], xbuf, copy_sems.at[0])
        cy = pltpu.make_async_copy(
            dy_hbm.at[pl.ds(row0, qm), :], dybuf, copy_sems.at[1])
        cx.start()
        cy.start()
        cx.wait()
        cy.wait()

        xv = xbuf[:, :]
        dyv = dybuf[:, :]
        mu = jnp.mean(xv, axis=1, keepdims=True)
        xc = xv - mu
        var = jnp.mean(xc * xc, axis=1, keepdims=True)
        rstd = lax.rsqrt(var + 1e-5)
        q = dyv * xc * rstd

        ones_row = jnp.ones((1, qm), jnp.float32)
        acc_ref[0:1, :] = jax.lax.dot_general(
            ones_row, q, (((1,), (0,)), ((), ())),
            preferred_element_type=jnp.float32)
        acc_ref[1:2, :] = jax.lax.dot_general(
            ones_row, dyv, (((1,), (0,)), ((), ())),
            preferred_element_type=jnp.float32)

        pl.semaphore_wait(barrier_sem, len(_OFFSETS))
        rdmas = []
        for k, o in enumerate(_OFFSETS):
            r = pltpu.make_async_remote_copy(
                src_ref=acc_ref,
                dst_ref=recv_ref.at[k],
                send_sem=send_sems.at[k],
                recv_sem=recv_sems.at[k],
                device_id=peer_of(o),
                device_id_type=pl.DeviceIdType.MESH,
            )
            r.start()
            rdmas.append(r)

        for r in rdmas:
            r.wait_recv()
        total_ref[:, :] = acc_ref[:, :] + jnp.sum(recv_ref[:, :, :], axis=0)

        out_copy = pltpu.make_async_copy(total_ref, out_hbm, out_sem)
        out_copy.start()
        out_copy.wait()
        for r in rdmas:
            r.wait_send()

    x = pltpu.with_memory_space_constraint(x, pltpu.MemorySpace.HBM)
    dy = pltpu.with_memory_space_constraint(dy, pltpu.MemorySpace.HBM)

    n_peer = len(_OFFSETS)
    return pl.pallas_call(
        body,
        out_shape=jax.ShapeDtypeStruct((2, d), jnp.float32),
        in_specs=[
            pl.BlockSpec(memory_space=pltpu.MemorySpace.HBM),
            pl.BlockSpec(memory_space=pltpu.MemorySpace.HBM),
        ],
        out_specs=pl.BlockSpec(memory_space=pltpu.MemorySpace.HBM),
        scratch_shapes=[
            pltpu.VMEM((qm, d), jnp.float32),
            pltpu.VMEM((qm, d), jnp.float32),
            pltpu.VMEM((2, d), jnp.float32),
            pltpu.VMEM((n_peer, 2, d), jnp.float32),
            pltpu.VMEM((2, d), jnp.float32),
            pltpu.SemaphoreType.DMA((2,)),
            pltpu.SemaphoreType.DMA,
            pltpu.SemaphoreType.DMA((n_peer,)),
            pltpu.SemaphoreType.DMA((n_peer,)),
        ],
        compiler_params=pltpu.CompilerParams(collective_id=0),
    )(x, dy)
